# baseline (speedup 1.0000x reference)
import functools

import jax
import jax.numpy as jnp
from jax import lax
from jax.experimental import pallas as pl
from jax.experimental.pallas import tpu as pltpu
from jax.experimental.pallas import tpu_sc as plsc

B = 50000
D = 3072
N = 1024
NC, NS, L = 2, 16, 16
NW = NC * NS
R = 1568
LAST_R = B - (NW - 1) * R
NCHUNK = N // L
CAP = N + L


def _body(memX, memy, memt, Xin, yin, tin, inds, outX, outy, outt,
          inds_v, table, s16, jlist, dlist, rows, yall, tall, yrange,
          trange, sem, csem):
    wid = lax.axis_index("s") * NC + lax.axis_index("c")
    lo = wid * R
    is_last = wid == NW - 1
    iota = lax.iota(jnp.int32, L)

    @pl.when(jnp.logical_not(is_last))
    def _():
        pltpu.async_copy(memX.at[pl.ds(lo, R)], outX.at[pl.ds(lo, R)], csem)

    @pl.when(is_last)
    def _():
        pltpu.async_copy(memX.at[pl.ds(lo, LAST_R)],
                         outX.at[pl.ds(lo, LAST_R)], csem)

    pltpu.sync_copy(inds, inds_v)
    pltpu.sync_copy(yin, yall)
    pltpu.sync_copy(tin, tall)

    @pl.when(jnp.logical_not(is_last))
    def _():
        pltpu.sync_copy(memy.at[pl.ds(lo, R)], yrange.at[pl.ds(0, R)])
        pltpu.sync_copy(memt.at[pl.ds(lo, R)], trange.at[pl.ds(0, R)])

    @pl.when(is_last)
    def _():
        pltpu.sync_copy(memy.at[pl.ds(lo, LAST_R)], yrange.at[pl.ds(0, LAST_R)])
        pltpu.sync_copy(memt.at[pl.ds(lo, LAST_R)], trange.at[pl.ds(0, LAST_R)])

    def init_tab(i, _):
        table[pl.ds(i * L, L)] = jnp.full((L,), -1, jnp.int32)
        return 0
    lax.fori_loop(0, R // L, init_tab, 0)

    def pass_a(c, _):
        iv = inds_v[pl.ds(c * L, L)]
        jv = iota + c * L
        s16[...] = iv
        dup = jnp.zeros((L,), jnp.int32)
        for s in range(1, L):
            nxt = plsc.load_gather(s16, [jnp.minimum(iota + s, L - 1)])
            valid = (iota + s) <= (L - 1)
            dup = jnp.where(jnp.logical_and(valid, nxt == iv), 1, dup)
        winlane = dup == 0
        local = iv - lo
        inr = jnp.logical_and(local >= 0, local < R)
        localc = jnp.clip(local, 0, R - 1)
        plsc.store_scatter(table, [localc], jv,
                           mask=jnp.logical_and(winlane, inr))
        return 0
    lax.fori_loop(0, NCHUNK, pass_a, 0)

    def pass_b(c, cnt):
        iv = inds_v[pl.ds(c * L, L)]
        jv = iota + c * L
        local = iv - lo
        inr = jnp.logical_and(local >= 0, local < R)
        localc = jnp.clip(local, 0, R - 1)
        tv = plsc.load_gather(table, [localc])
        win = jnp.logical_and(inr, tv == jv)
        wc = plsc.cumsum(win.astype(jnp.int32))
        pos = jnp.clip(cnt + wc - 1, 0, CAP - 1)
        plsc.store_scatter(jlist, [pos], jv, mask=win)
        plsc.store_scatter(dlist, [pos], iv, mask=win)
        return cnt + jnp.max(wc)
    cnt = lax.fori_loop(0, NCHUNK, pass_b, jnp.int32(0))

    lastp = jnp.full((L,), jnp.clip(cnt - 1, 0, CAP - 1), jnp.int32)
    jlast = plsc.load_gather(jlist, [lastp])
    dlast = plsc.load_gather(dlist, [lastp])
    padp = jnp.clip(cnt + iota, 0, CAP - 1)
    plsc.store_scatter(jlist, [padp], jlast)
    plsc.store_scatter(dlist, [padp], dlast)

    @pl.when(jnp.logical_not(is_last))
    def _():
        pltpu.make_async_copy(memX.at[pl.ds(lo, R)],
                              outX.at[pl.ds(lo, R)], csem).wait()

    @pl.when(is_last)
    def _():
        pltpu.make_async_copy(memX.at[pl.ds(lo, LAST_R)],
                              outX.at[pl.ds(lo, LAST_R)], csem).wait()

    nch = (cnt + L - 1) // L

    def scatter_chunk(i, _):
        jv = jlist[pl.ds(i * L, L)]
        dv = dlist[pl.ds(i * L, L)]
        pltpu.async_copy(Xin.at[jv], rows, sem).wait()
        pltpu.async_copy(rows, outX.at[dv], sem).wait()
        ldv = dv - lo
        plsc.store_scatter(yrange, [ldv], plsc.load_gather(yall, [jv]))
        plsc.store_scatter(trange, [ldv], plsc.load_gather(tall, [jv]))
        return 0
    lax.fori_loop(0, nch, scatter_chunk, 0)

    @pl.when(jnp.logical_not(is_last))
    def _():
        pltpu.sync_copy(yrange.at[pl.ds(0, R)], outy.at[pl.ds(lo, R)])
        pltpu.sync_copy(trange.at[pl.ds(0, R)], outt.at[pl.ds(lo, R)])

    @pl.when(is_last)
    def _():
        pltpu.sync_copy(yrange.at[pl.ds(0, LAST_R)], outy.at[pl.ds(lo, LAST_R)])
        pltpu.sync_copy(trange.at[pl.ds(0, LAST_R)], outt.at[pl.ds(lo, LAST_R)])


_sc_call = functools.partial(
    pl.kernel,
    out_type=(
        jax.ShapeDtypeStruct((B, D), jnp.float32),
        jax.ShapeDtypeStruct((B,), jnp.float32),
        jax.ShapeDtypeStruct((B,), jnp.int32),
    ),
    mesh=plsc.VectorSubcoreMesh(core_axis_name="c", subcore_axis_name="s"),
    compiler_params=pltpu.CompilerParams(needs_layout_passes=False),
    scratch_types=[
        pltpu.VMEM((N,), jnp.int32),
        pltpu.VMEM((R,), jnp.int32),
        pltpu.VMEM((L,), jnp.int32),
        pltpu.VMEM((CAP,), jnp.int32),
        pltpu.VMEM((CAP,), jnp.int32),
        pltpu.VMEM((L, D), jnp.float32),
        pltpu.VMEM((N,), jnp.float32),
        pltpu.VMEM((N,), jnp.int32),
        pltpu.VMEM((R,), jnp.float32),
        pltpu.VMEM((R,), jnp.int32),
        pltpu.SemaphoreType.DMA,
        pltpu.SemaphoreType.DMA,
    ],
)(_body)


def kernel(mem_X, mem_y, mem_task_ids, X, y, task_ids, inds):
    out_X, out_y, out_t = _sc_call(
        mem_X.reshape(B, D), mem_y, mem_task_ids,
        X.reshape(N, D), y, task_ids.astype(jnp.int32),
        inds.astype(jnp.int32))
    return (out_X.reshape(mem_X.shape), out_y, out_t)


# speedup vs baseline: 12.7580x; 12.7580x over previous
import functools

import jax
import jax.numpy as jnp
from jax import lax
from jax.experimental import pallas as pl
from jax.experimental.pallas import tpu as pltpu
from jax.experimental.pallas import tpu_sc as plsc

B = 50000
D = 3072
N = 1024
NC, NS, L = 2, 16, 16
NW = NC * NS
R = 1568
LAST_R = B - (NW - 1) * R
NCHUNK = N // L
CAP = N + L


def _body(memX, memy, memt, Xin, yin, tin, inds, outX, outy, outt,
          inds_v, table, s16, jlist, dlist, rows, cbuf, yall, tall, yrange,
          trange, sem, gsem0, gsem1, ssem0, ssem1):
    wid = lax.axis_index("s") * NC + lax.axis_index("c")
    lo = wid * R
    is_last = wid == NW - 1
    iota = lax.iota(jnp.int32, L)

    pltpu.sync_copy(inds, inds_v)
    pltpu.sync_copy(yin, yall)
    pltpu.sync_copy(tin, tall)

    @pl.when(jnp.logical_not(is_last))
    def _():
        pltpu.sync_copy(memy.at[pl.ds(lo, R)], yrange.at[pl.ds(0, R)])
        pltpu.sync_copy(memt.at[pl.ds(lo, R)], trange.at[pl.ds(0, R)])

    @pl.when(is_last)
    def _():
        pltpu.sync_copy(memy.at[pl.ds(lo, LAST_R)], yrange.at[pl.ds(0, LAST_R)])
        pltpu.sync_copy(memt.at[pl.ds(lo, LAST_R)], trange.at[pl.ds(0, LAST_R)])

    nch_copy = jnp.where(is_last, LAST_R // L, R // L)

    pltpu.async_copy(memX.at[pl.ds(lo, L)], rows, gsem0)
    pltpu.async_copy(memX.at[pl.ds(lo + L, L)], cbuf, gsem1)

    def copy_chunk(i, _):
        a = lo + i * L

        def turn(buf, gsem, ssem):
            pltpu.make_async_copy(memX.at[pl.ds(a, L)], buf, gsem).wait()
            pltpu.async_copy(buf, outX.at[pl.ds(a, L)], ssem)

            @pl.when(i + 2 < nch_copy)
            def _():
                pltpu.make_async_copy(buf, outX.at[pl.ds(a, L)], ssem).wait()
                pltpu.async_copy(memX.at[pl.ds(a + 2 * L, L)], buf, gsem)

        @pl.when(i % 2 == 0)
        def _():
            turn(rows, gsem0, ssem0)

        @pl.when(i % 2 == 1)
        def _():
            turn(cbuf, gsem1, ssem1)

        return 0
    lax.fori_loop(0, nch_copy, copy_chunk, 0)
    pltpu.make_async_copy(rows, outX.at[pl.ds(lo, L)], ssem0).wait()
    pltpu.make_async_copy(cbuf, outX.at[pl.ds(lo, L)], ssem1).wait()

    def init_tab(i, _):
        table[pl.ds(i * L, L)] = jnp.full((L,), -1, jnp.int32)
        return 0
    lax.fori_loop(0, R // L, init_tab, 0)

    def pass_a(c, _):
        iv = inds_v[pl.ds(c * L, L)]
        jv = iota + c * L
        s16[...] = iv
        dup = jnp.zeros((L,), jnp.int32)
        for s in range(1, L):
            nxt = plsc.load_gather(s16, [jnp.minimum(iota + s, L - 1)])
            valid = (iota + s) <= (L - 1)
            dup = jnp.where(jnp.logical_and(valid, nxt == iv), 1, dup)
        winlane = dup == 0
        local = iv - lo
        inr = jnp.logical_and(local >= 0, local < R)
        localc = jnp.clip(local, 0, R - 1)
        plsc.store_scatter(table, [localc], jv,
                           mask=jnp.logical_and(winlane, inr))
        return 0
    lax.fori_loop(0, NCHUNK, pass_a, 0)

    def pass_b(c, cnt):
        iv = inds_v[pl.ds(c * L, L)]
        jv = iota + c * L
        local = iv - lo
        inr = jnp.logical_and(local >= 0, local < R)
        localc = jnp.clip(local, 0, R - 1)
        tv = plsc.load_gather(table, [localc])
        win = jnp.logical_and(inr, tv == jv)
        wc = plsc.cumsum(win.astype(jnp.int32))
        pos = jnp.clip(cnt + wc - 1, 0, CAP - 1)
        plsc.store_scatter(jlist, [pos], jv, mask=win)
        plsc.store_scatter(dlist, [pos], iv, mask=win)
        return cnt + jnp.max(wc)
    cnt = lax.fori_loop(0, NCHUNK, pass_b, jnp.int32(0))

    lastp = jnp.full((L,), jnp.clip(cnt - 1, 0, CAP - 1), jnp.int32)
    jlast = plsc.load_gather(jlist, [lastp])
    dlast = plsc.load_gather(dlist, [lastp])
    padp = jnp.clip(cnt + iota, 0, CAP - 1)
    plsc.store_scatter(jlist, [padp], jlast)
    plsc.store_scatter(dlist, [padp], dlast)

    nch = (cnt + L - 1) // L

    def scatter_chunk(i, _):
        jv = jlist[pl.ds(i * L, L)]
        dv = dlist[pl.ds(i * L, L)]
        pltpu.async_copy(Xin.at[jv], rows, sem).wait()
        pltpu.async_copy(rows, outX.at[dv], sem).wait()
        ldv = dv - lo
        plsc.store_scatter(yrange, [ldv], plsc.load_gather(yall, [jv]))
        plsc.store_scatter(trange, [ldv], plsc.load_gather(tall, [jv]))
        return 0
    lax.fori_loop(0, nch, scatter_chunk, 0)

    @pl.when(jnp.logical_not(is_last))
    def _():
        pltpu.sync_copy(yrange.at[pl.ds(0, R)], outy.at[pl.ds(lo, R)])
        pltpu.sync_copy(trange.at[pl.ds(0, R)], outt.at[pl.ds(lo, R)])

    @pl.when(is_last)
    def _():
        pltpu.sync_copy(yrange.at[pl.ds(0, LAST_R)], outy.at[pl.ds(lo, LAST_R)])
        pltpu.sync_copy(trange.at[pl.ds(0, LAST_R)], outt.at[pl.ds(lo, LAST_R)])


_sc_call = functools.partial(
    pl.kernel,
    out_type=(
        jax.ShapeDtypeStruct((B, D), jnp.float32),
        jax.ShapeDtypeStruct((B,), jnp.float32),
        jax.ShapeDtypeStruct((B,), jnp.int32),
    ),
    mesh=plsc.VectorSubcoreMesh(core_axis_name="c", subcore_axis_name="s"),
    compiler_params=pltpu.CompilerParams(needs_layout_passes=False),
    scratch_types=[
        pltpu.VMEM((N,), jnp.int32),
        pltpu.VMEM((R,), jnp.int32),
        pltpu.VMEM((L,), jnp.int32),
        pltpu.VMEM((CAP,), jnp.int32),
        pltpu.VMEM((CAP,), jnp.int32),
        pltpu.VMEM((L, D), jnp.float32),
        pltpu.VMEM((L, D), jnp.float32),
        pltpu.VMEM((N,), jnp.float32),
        pltpu.VMEM((N,), jnp.int32),
        pltpu.VMEM((R,), jnp.float32),
        pltpu.VMEM((R,), jnp.int32),
        pltpu.SemaphoreType.DMA,
        pltpu.SemaphoreType.DMA,
        pltpu.SemaphoreType.DMA,
        pltpu.SemaphoreType.DMA,
        pltpu.SemaphoreType.DMA,
    ],
)(_body)


def kernel(mem_X, mem_y, mem_task_ids, X, y, task_ids, inds):
    out_X, out_y, out_t = _sc_call(
        mem_X.reshape(B, D), mem_y, mem_task_ids,
        X.reshape(N, D), y, task_ids.astype(jnp.int32),
        inds.astype(jnp.int32))
    return (out_X.reshape(mem_X.shape), out_y, out_t)


# speedup vs baseline: 12.8117x; 1.0042x over previous
import functools

import jax
import jax.numpy as jnp
from jax import lax
from jax.experimental import pallas as pl
from jax.experimental.pallas import tpu as pltpu
from jax.experimental.pallas import tpu_sc as plsc

B = 50000
D = 3072
N = 1024
NC, NS, L = 2, 16, 16
NW = NC * NS
R = 1568
LAST_R = B - (NW - 1) * R
NCHUNK = N // L
CAP = N + L
G = 8
NBUF = 4


def _body(memX, memy, memt, Xin, yin, tin, inds, outX, outy, outt,
          inds_v, table, s16, jlist, dlist, dlist2, buf0, buf1, buf2, buf3,
          yall, tall, yrange, trange, sem,
          gsem0, gsem1, gsem2, gsem3, ssem0, ssem1, ssem2, ssem3):
    wid = lax.axis_index("s") * NC + lax.axis_index("c")
    lo = wid * R
    is_last = wid == NW - 1
    iota = lax.iota(jnp.int32, L)
    bufs = (buf0, buf1, buf2, buf3)
    gsems = (gsem0, gsem1, gsem2, gsem3)
    ssems = (ssem0, ssem1, ssem2, ssem3)

    pltpu.sync_copy(inds, inds_v)
    pltpu.sync_copy(yin, yall)
    pltpu.sync_copy(tin, tall)

    @pl.when(jnp.logical_not(is_last))
    def _():
        pltpu.sync_copy(memy.at[pl.ds(lo, R)], yrange.at[pl.ds(0, R)])
        pltpu.sync_copy(memt.at[pl.ds(lo, R)], trange.at[pl.ds(0, R)])

    @pl.when(is_last)
    def _():
        pltpu.sync_copy(memy.at[pl.ds(lo, LAST_R)], yrange.at[pl.ds(0, LAST_R)])
        pltpu.sync_copy(memt.at[pl.ds(lo, LAST_R)], trange.at[pl.ds(0, LAST_R)])

    nch_copy = jnp.where(is_last, LAST_R // G, R // G)

    for k in range(NBUF):
        pltpu.async_copy(memX.at[pl.ds(lo + k * G, G)], bufs[k], gsems[k])

    def copy_chunk(i, _):
        a = lo + i * G

        def turn(buf, gsem, ssem):
            pltpu.make_async_copy(memX.at[pl.ds(a, G)], buf, gsem).wait()
            pltpu.async_copy(buf, outX.at[pl.ds(a, G)], ssem)

            @pl.when(i + NBUF < nch_copy)
            def _():
                pltpu.make_async_copy(buf, outX.at[pl.ds(a, G)], ssem).wait()
                pltpu.async_copy(memX.at[pl.ds(a + NBUF * G, G)], buf, gsem)

        for k in range(NBUF):
            @pl.when(i % NBUF == k)
            def _(k=k):
                turn(bufs[k], gsems[k], ssems[k])

        return 0
    lax.fori_loop(0, nch_copy, copy_chunk, 0)
    for k in range(NBUF):
        pltpu.make_async_copy(bufs[k], outX.at[pl.ds(lo, G)], ssems[k]).wait()

    def init_tab(i, _):
        table[pl.ds(i * L, L)] = jnp.full((L,), -1, jnp.int32)
        return 0
    lax.fori_loop(0, R // L, init_tab, 0)

    def pass_a(c, _):
        iv = inds_v[pl.ds(c * L, L)]
        jv = iota + c * L
        s16[...] = iv
        dup = jnp.zeros((L,), jnp.int32)
        for s in range(1, L):
            nxt = plsc.load_gather(s16, [jnp.minimum(iota + s, L - 1)])
            valid = (iota + s) <= (L - 1)
            dup = jnp.where(jnp.logical_and(valid, nxt == iv), 1, dup)
        winlane = dup == 0
        local = iv - lo
        inr = jnp.logical_and(local >= 0, local < R)
        localc = jnp.clip(local, 0, R - 1)
        plsc.store_scatter(table, [localc], jv,
                           mask=jnp.logical_and(winlane, inr))
        return 0
    lax.fori_loop(0, NCHUNK, pass_a, 0)

    def pass_b(c, cnt):
        iv = inds_v[pl.ds(c * L, L)]
        jv = iota + c * L
        local = iv - lo
        inr = jnp.logical_and(local >= 0, local < R)
        localc = jnp.clip(local, 0, R - 1)
        tv = plsc.load_gather(table, [localc])
        win = jnp.logical_and(inr, tv == jv)
        wc = plsc.cumsum(win.astype(jnp.int32))
        pos = jnp.clip(cnt + wc - 1, 0, CAP - 1)
        plsc.store_scatter(jlist, [pos], jv, mask=win)
        plsc.store_scatter(dlist, [pos], iv, mask=win)
        plsc.store_scatter(dlist2, [pos // G, pos - (pos // G) * G], iv,
                           mask=win)
        return cnt + jnp.max(wc)
    cnt = lax.fori_loop(0, NCHUNK, pass_b, jnp.int32(0))

    lastp = jnp.full((L,), jnp.clip(cnt - 1, 0, CAP - 1), jnp.int32)
    jlast = plsc.load_gather(jlist, [lastp])
    dlast = plsc.load_gather(dlist, [lastp])
    padp = jnp.clip(cnt + iota, 0, CAP - 1)
    plsc.store_scatter(jlist, [padp], jlast)
    plsc.store_scatter(dlist, [padp], dlast)
    plsc.store_scatter(dlist2, [padp // G, padp - (padp // G) * G], dlast)

    nchu = (cnt + G - 1) // G

    @pl.when(nchu > 0)
    def _():
        pltpu.async_copy(Xin.at[jlist.at[pl.ds(0, G)]], buf0, gsem0)

    @pl.when(nchu > 1)
    def _():
        pltpu.async_copy(Xin.at[jlist.at[pl.ds(G, G)]], buf1, gsem1)

    def update_chunk(i, _):
        def turn(buf, gsem, ssem):
            pltpu.make_async_copy(Xin.at[jlist.at[pl.ds(i * G, G)]],
                                  buf, gsem).wait()
            pltpu.async_copy(buf, outX.at[dlist2.at[i]], ssem)
            pltpu.make_async_copy(buf, outX.at[dlist2.at[i]], ssem).wait()

            @pl.when(i + 2 < nchu)
            def _():
                pltpu.async_copy(Xin.at[jlist.at[pl.ds((i + 2) * G, G)]],
                                 buf, gsem)

        @pl.when(i % 2 == 0)
        def _():
            turn(buf0, gsem0, ssem0)

        @pl.when(i % 2 == 1)
        def _():
            turn(buf1, gsem1, ssem1)

        return 0
    lax.fori_loop(0, nchu, update_chunk, 0)

    nch16 = (cnt + L - 1) // L

    def yt_chunk(i, _):
        jv = jlist[pl.ds(i * L, L)]
        dv = dlist[pl.ds(i * L, L)]
        ldv = dv - lo
        plsc.store_scatter(yrange, [ldv], plsc.load_gather(yall, [jv]))
        plsc.store_scatter(trange, [ldv], plsc.load_gather(tall, [jv]))
        return 0
    lax.fori_loop(0, nch16, yt_chunk, 0)

    @pl.when(jnp.logical_not(is_last))
    def _():
        pltpu.sync_copy(yrange.at[pl.ds(0, R)], outy.at[pl.ds(lo, R)])
        pltpu.sync_copy(trange.at[pl.ds(0, R)], outt.at[pl.ds(lo, R)])

    @pl.when(is_last)
    def _():
        pltpu.sync_copy(yrange.at[pl.ds(0, LAST_R)], outy.at[pl.ds(lo, LAST_R)])
        pltpu.sync_copy(trange.at[pl.ds(0, LAST_R)], outt.at[pl.ds(lo, LAST_R)])


_sc_call = functools.partial(
    pl.kernel,
    out_type=(
        jax.ShapeDtypeStruct((B, D), jnp.float32),
        jax.ShapeDtypeStruct((B,), jnp.float32),
        jax.ShapeDtypeStruct((B,), jnp.int32),
    ),
    mesh=plsc.VectorSubcoreMesh(core_axis_name="c", subcore_axis_name="s"),
    compiler_params=pltpu.CompilerParams(needs_layout_passes=False),
    scratch_types=[
        pltpu.VMEM((N,), jnp.int32),
        pltpu.VMEM((R,), jnp.int32),
        pltpu.VMEM((L,), jnp.int32),
        pltpu.VMEM((CAP,), jnp.int32),
        pltpu.VMEM((CAP,), jnp.int32),
        pltpu.VMEM((CAP // G, G), jnp.int32),
        pltpu.VMEM((G, D), jnp.float32),
        pltpu.VMEM((G, D), jnp.float32),
        pltpu.VMEM((G, D), jnp.float32),
        pltpu.VMEM((G, D), jnp.float32),
        pltpu.VMEM((N,), jnp.float32),
        pltpu.VMEM((N,), jnp.int32),
        pltpu.VMEM((R,), jnp.float32),
        pltpu.VMEM((R,), jnp.int32),
        pltpu.SemaphoreType.DMA,
        pltpu.SemaphoreType.DMA,
        pltpu.SemaphoreType.DMA,
        pltpu.SemaphoreType.DMA,
        pltpu.SemaphoreType.DMA,
        pltpu.SemaphoreType.DMA,
        pltpu.SemaphoreType.DMA,
        pltpu.SemaphoreType.DMA,
        pltpu.SemaphoreType.DMA,
    ],
)(_body)


def kernel(mem_X, mem_y, mem_task_ids, X, y, task_ids, inds):
    out_X, out_y, out_t = _sc_call(
        mem_X.reshape(B, D), mem_y, mem_task_ids,
        X.reshape(N, D), y, task_ids.astype(jnp.int32),
        inds.astype(jnp.int32))
    return (out_X.reshape(mem_X.shape), out_y, out_t)


# speedup vs baseline: 13.0688x; 1.0201x over previous
import functools

import jax
import jax.numpy as jnp
from jax import lax
from jax.experimental import pallas as pl
from jax.experimental.pallas import tpu as pltpu
from jax.experimental.pallas import tpu_sc as plsc

B = 50000
D = 3072
N = 1024
NC, NS, L = 2, 16, 16
NW = NC * NS
R = 1568
LAST_R = B - (NW - 1) * R
NCHUNK = N // L
CAP = N + L
G = 8
NBUF = 2


def _body(memX, memy, memt, Xin, yin, tin, inds, outX, outy, outt,
          inds_v, table, s16, jlist, dlist, dlist2, spbuf, ubuf0, ubuf1,
          yall, tall, yrange, trange, sem,
          gsem0, gsem1, gsem2, gsem3, ssem0, ssem1, ssem2, ssem3):
    sid = lax.axis_index("s")
    wid = sid * NC + lax.axis_index("c")
    lo = wid * R
    is_last = wid == NW - 1
    iota = lax.iota(jnp.int32, L)
    bufs = tuple(spbuf.at[sid, k] for k in range(NBUF))
    gsems = (gsem0, gsem1, gsem2, gsem3)
    ssems = (ssem0, ssem1, ssem2, ssem3)

    pltpu.sync_copy(inds, inds_v)
    pltpu.sync_copy(yin, yall)
    pltpu.sync_copy(tin, tall)

    @pl.when(jnp.logical_not(is_last))
    def _():
        pltpu.sync_copy(memy.at[pl.ds(lo, R)], yrange.at[pl.ds(0, R)])
        pltpu.sync_copy(memt.at[pl.ds(lo, R)], trange.at[pl.ds(0, R)])

    @pl.when(is_last)
    def _():
        pltpu.sync_copy(memy.at[pl.ds(lo, LAST_R)], yrange.at[pl.ds(0, LAST_R)])
        pltpu.sync_copy(memt.at[pl.ds(lo, LAST_R)], trange.at[pl.ds(0, LAST_R)])

    nch_copy = jnp.where(is_last, LAST_R // G, R // G)

    for k in range(NBUF):
        pltpu.async_copy(memX.at[pl.ds(lo + k * G, G)], bufs[k], gsems[k])

    def copy_chunk(i, _):
        a = lo + i * G

        def turn(buf, gsem, ssem):
            pltpu.make_async_copy(memX.at[pl.ds(a, G)], buf, gsem).wait()
            pltpu.async_copy(buf, outX.at[pl.ds(a, G)], ssem)

            @pl.when(i + NBUF < nch_copy)
            def _():
                pltpu.make_async_copy(buf, outX.at[pl.ds(a, G)], ssem).wait()
                pltpu.async_copy(memX.at[pl.ds(a + NBUF * G, G)], buf, gsem)

        for k in range(NBUF):
            @pl.when(i % NBUF == k)
            def _(k=k):
                turn(bufs[k], gsems[k], ssems[k])

        return 0
    lax.fori_loop(0, nch_copy, copy_chunk, 0)
    for k in range(NBUF):
        pltpu.make_async_copy(bufs[k], outX.at[pl.ds(lo, G)], ssems[k]).wait()

    def init_tab(i, _):
        table[pl.ds(i * L, L)] = jnp.full((L,), -1, jnp.int32)
        return 0
    lax.fori_loop(0, R // L, init_tab, 0)

    def pass_a(c, _):
        iv = inds_v[pl.ds(c * L, L)]
        jv = iota + c * L
        s16[...] = iv
        dup = jnp.zeros((L,), jnp.int32)
        for s in range(1, L):
            nxt = plsc.load_gather(s16, [jnp.minimum(iota + s, L - 1)])
            valid = (iota + s) <= (L - 1)
            dup = jnp.where(jnp.logical_and(valid, nxt == iv), 1, dup)
        winlane = dup == 0
        local = iv - lo
        inr = jnp.logical_and(local >= 0, local < R)
        localc = jnp.clip(local, 0, R - 1)
        plsc.store_scatter(table, [localc], jv,
                           mask=jnp.logical_and(winlane, inr))
        return 0
    lax.fori_loop(0, NCHUNK, pass_a, 0)

    def pass_b(c, cnt):
        iv = inds_v[pl.ds(c * L, L)]
        jv = iota + c * L
        local = iv - lo
        inr = jnp.logical_and(local >= 0, local < R)
        localc = jnp.clip(local, 0, R - 1)
        tv = plsc.load_gather(table, [localc])
        win = jnp.logical_and(inr, tv == jv)
        wc = plsc.cumsum(win.astype(jnp.int32))
        pos = jnp.clip(cnt + wc - 1, 0, CAP - 1)
        plsc.store_scatter(jlist, [pos], jv, mask=win)
        plsc.store_scatter(dlist, [pos], iv, mask=win)
        plsc.store_scatter(dlist2, [pos // G, pos - (pos // G) * G], iv,
                           mask=win)
        return cnt + jnp.max(wc)
    cnt = lax.fori_loop(0, NCHUNK, pass_b, jnp.int32(0))

    lastp = jnp.full((L,), jnp.clip(cnt - 1, 0, CAP - 1), jnp.int32)
    jlast = plsc.load_gather(jlist, [lastp])
    dlast = plsc.load_gather(dlist, [lastp])
    padp = jnp.clip(cnt + iota, 0, CAP - 1)
    plsc.store_scatter(jlist, [padp], jlast)
    plsc.store_scatter(dlist, [padp], dlast)
    plsc.store_scatter(dlist2, [padp // G, padp - (padp // G) * G], dlast)

    nchu = (cnt + G - 1) // G

    @pl.when(nchu > 0)
    def _():
        pltpu.async_copy(Xin.at[jlist.at[pl.ds(0, G)]], ubuf0, gsem0)

    @pl.when(nchu > 1)
    def _():
        pltpu.async_copy(Xin.at[jlist.at[pl.ds(G, G)]], ubuf1, gsem1)

    def update_chunk(i, _):
        def turn(buf, gsem, ssem):
            pltpu.make_async_copy(Xin.at[jlist.at[pl.ds(i * G, G)]],
                                  buf, gsem).wait()
            pltpu.async_copy(buf, outX.at[dlist2.at[i]], ssem)
            pltpu.make_async_copy(buf, outX.at[dlist2.at[i]], ssem).wait()

            @pl.when(i + 2 < nchu)
            def _():
                pltpu.async_copy(Xin.at[jlist.at[pl.ds((i + 2) * G, G)]],
                                 buf, gsem)

        @pl.when(i % 2 == 0)
        def _():
            turn(ubuf0, gsem0, ssem0)

        @pl.when(i % 2 == 1)
        def _():
            turn(ubuf1, gsem1, ssem1)

        return 0
    lax.fori_loop(0, nchu, update_chunk, 0)

    nch16 = (cnt + L - 1) // L

    def yt_chunk(i, _):
        jv = jlist[pl.ds(i * L, L)]
        dv = dlist[pl.ds(i * L, L)]
        ldv = dv - lo
        plsc.store_scatter(yrange, [ldv], plsc.load_gather(yall, [jv]))
        plsc.store_scatter(trange, [ldv], plsc.load_gather(tall, [jv]))
        return 0
    lax.fori_loop(0, nch16, yt_chunk, 0)

    @pl.when(jnp.logical_not(is_last))
    def _():
        pltpu.sync_copy(yrange.at[pl.ds(0, R)], outy.at[pl.ds(lo, R)])
        pltpu.sync_copy(trange.at[pl.ds(0, R)], outt.at[pl.ds(lo, R)])

    @pl.when(is_last)
    def _():
        pltpu.sync_copy(yrange.at[pl.ds(0, LAST_R)], outy.at[pl.ds(lo, LAST_R)])
        pltpu.sync_copy(trange.at[pl.ds(0, LAST_R)], outt.at[pl.ds(lo, LAST_R)])


_sc_call = functools.partial(
    pl.kernel,
    out_type=(
        jax.ShapeDtypeStruct((B, D), jnp.float32),
        jax.ShapeDtypeStruct((B,), jnp.float32),
        jax.ShapeDtypeStruct((B,), jnp.int32),
    ),
    mesh=plsc.VectorSubcoreMesh(core_axis_name="c", subcore_axis_name="s"),
    compiler_params=pltpu.CompilerParams(needs_layout_passes=False),
    scratch_types=[
        pltpu.VMEM((N,), jnp.int32),
        pltpu.VMEM((R,), jnp.int32),
        pltpu.VMEM((L,), jnp.int32),
        pltpu.VMEM((CAP,), jnp.int32),
        pltpu.VMEM((CAP,), jnp.int32),
        pltpu.VMEM((CAP // G, G), jnp.int32),
        pltpu.VMEM_SHARED((NS, NBUF, G, D), jnp.float32),
        pltpu.VMEM((G, D), jnp.float32),
        pltpu.VMEM((G, D), jnp.float32),
        pltpu.VMEM((N,), jnp.float32),
        pltpu.VMEM((N,), jnp.int32),
        pltpu.VMEM((R,), jnp.float32),
        pltpu.VMEM((R,), jnp.int32),
        pltpu.SemaphoreType.DMA,
        pltpu.SemaphoreType.DMA,
        pltpu.SemaphoreType.DMA,
        pltpu.SemaphoreType.DMA,
        pltpu.SemaphoreType.DMA,
        pltpu.SemaphoreType.DMA,
        pltpu.SemaphoreType.DMA,
        pltpu.SemaphoreType.DMA,
        pltpu.SemaphoreType.DMA,
    ],
)(_body)


def kernel(mem_X, mem_y, mem_task_ids, X, y, task_ids, inds):
    out_X, out_y, out_t = _sc_call(
        mem_X.reshape(B, D), mem_y, mem_task_ids,
        X.reshape(N, D), y, task_ids.astype(jnp.int32),
        inds.astype(jnp.int32))
    return (out_X.reshape(mem_X.shape), out_y, out_t)


# speedup vs baseline: 13.1125x; 1.0033x over previous
import functools

import jax
import jax.numpy as jnp
from jax import lax
from jax.experimental import pallas as pl
from jax.experimental.pallas import tpu as pltpu
from jax.experimental.pallas import tpu_sc as plsc

B = 50000
D = 3072
N = 1024
NC, NS, L = 2, 16, 16
NW = NC * NS
R = 1568
LAST_R = B - (NW - 1) * R
NCHUNK = N // L
CAP = N + L
G = 8
NBUF = 2


def _body(memX, memy, memt, Xin, yin, tin, inds, outX, outy, outt,
          inds_v, table, s16, jlist, dlist, dlist2, spbuf, ubuf0, ubuf1,
          yall, tall, yrange, trange, sem,
          gsem0, gsem1, gsem2, gsem3, ssem0, ssem1, ssem2, ssem3):
    sid = lax.axis_index("s")
    wid = sid * NC + lax.axis_index("c")
    lo = wid * R
    is_last = wid == NW - 1
    iota = lax.iota(jnp.int32, L)
    bufs = tuple(spbuf.at[sid, k] for k in range(NBUF))
    gsems = (gsem0, gsem1, gsem2, gsem3)
    ssems = (ssem0, ssem1, ssem2, ssem3)

    pltpu.sync_copy(inds, inds_v)
    pltpu.sync_copy(yin, yall)
    pltpu.sync_copy(tin, tall)

    @pl.when(jnp.logical_not(is_last))
    def _():
        pltpu.sync_copy(memy.at[pl.ds(lo, R)], yrange.at[pl.ds(0, R)])
        pltpu.sync_copy(memt.at[pl.ds(lo, R)], trange.at[pl.ds(0, R)])

    @pl.when(is_last)
    def _():
        pltpu.sync_copy(memy.at[pl.ds(lo, LAST_R)], yrange.at[pl.ds(0, LAST_R)])
        pltpu.sync_copy(memt.at[pl.ds(lo, LAST_R)], trange.at[pl.ds(0, LAST_R)])

    nhalf = jnp.where(is_last, LAST_R // (2 * G), R // (2 * G))
    half = nhalf * G
    loA = lo
    loB = lo + half
    bufsB = (ubuf0, ubuf1)
    gsemsB = (gsem2, gsem3)
    ssemsB = (ssem2, ssem3)

    for k in range(NBUF):
        pltpu.async_copy(memX.at[pl.ds(loA + k * G, G)], bufs[k], gsems[k])
        pltpu.async_copy(memX.at[pl.ds(loB + k * G, G)], bufsB[k], gsemsB[k])

    def copy_chunk(i, _):
        aA = loA + i * G
        aB = loB + i * G

        def turn(k):
            pltpu.make_async_copy(memX.at[pl.ds(aA, G)], bufs[k],
                                  gsems[k]).wait()
            pltpu.async_copy(bufs[k], outX.at[pl.ds(aA, G)], ssems[k])
            pltpu.make_async_copy(memX.at[pl.ds(aB, G)], bufsB[k],
                                  gsemsB[k]).wait()
            pltpu.async_copy(bufsB[k], outX.at[pl.ds(aB, G)], ssemsB[k])

            @pl.when(i + NBUF < nhalf)
            def _():
                pltpu.make_async_copy(bufs[k], outX.at[pl.ds(aA, G)],
                                      ssems[k]).wait()
                pltpu.async_copy(memX.at[pl.ds(aA + NBUF * G, G)],
                                 bufs[k], gsems[k])
                pltpu.make_async_copy(bufsB[k], outX.at[pl.ds(aB, G)],
                                      ssemsB[k]).wait()
                pltpu.async_copy(memX.at[pl.ds(aB + NBUF * G, G)],
                                 bufsB[k], gsemsB[k])

        for k in range(NBUF):
            @pl.when(i % NBUF == k)
            def _(k=k):
                turn(k)

        return 0
    lax.fori_loop(0, nhalf, copy_chunk, 0)
    for k in range(NBUF):
        pltpu.make_async_copy(bufs[k], outX.at[pl.ds(lo, G)], ssems[k]).wait()
        pltpu.make_async_copy(bufsB[k], outX.at[pl.ds(lo, G)],
                              ssemsB[k]).wait()

    def init_tab(i, _):
        table[pl.ds(i * L, L)] = jnp.full((L,), -1, jnp.int32)
        return 0
    lax.fori_loop(0, R // L, init_tab, 0)

    def pass_a(c, _):
        iv = inds_v[pl.ds(c * L, L)]
        jv = iota + c * L
        s16[...] = iv
        dup = jnp.zeros((L,), jnp.int32)
        for s in range(1, L):
            nxt = plsc.load_gather(s16, [jnp.minimum(iota + s, L - 1)])
            valid = (iota + s) <= (L - 1)
            dup = jnp.where(jnp.logical_and(valid, nxt == iv), 1, dup)
        winlane = dup == 0
        local = iv - lo
        inr = jnp.logical_and(local >= 0, local < R)
        localc = jnp.clip(local, 0, R - 1)
        plsc.store_scatter(table, [localc], jv,
                           mask=jnp.logical_and(winlane, inr))
        return 0
    lax.fori_loop(0, NCHUNK, pass_a, 0)

    def pass_b(c, cnt):
        iv = inds_v[pl.ds(c * L, L)]
        jv = iota + c * L
        local = iv - lo
        inr = jnp.logical_and(local >= 0, local < R)
        localc = jnp.clip(local, 0, R - 1)
        tv = plsc.load_gather(table, [localc])
        win = jnp.logical_and(inr, tv == jv)
        wc = plsc.cumsum(win.astype(jnp.int32))
        pos = jnp.clip(cnt + wc - 1, 0, CAP - 1)
        plsc.store_scatter(jlist, [pos], jv, mask=win)
        plsc.store_scatter(dlist, [pos], iv, mask=win)
        plsc.store_scatter(dlist2, [pos // G, pos - (pos // G) * G], iv,
                           mask=win)
        return cnt + jnp.max(wc)
    cnt = lax.fori_loop(0, NCHUNK, pass_b, jnp.int32(0))

    lastp = jnp.full((L,), jnp.clip(cnt - 1, 0, CAP - 1), jnp.int32)
    jlast = plsc.load_gather(jlist, [lastp])
    dlast = plsc.load_gather(dlist, [lastp])
    padp = jnp.clip(cnt + iota, 0, CAP - 1)
    plsc.store_scatter(jlist, [padp], jlast)
    plsc.store_scatter(dlist, [padp], dlast)
    plsc.store_scatter(dlist2, [padp // G, padp - (padp // G) * G], dlast)

    nchu = (cnt + G - 1) // G

    @pl.when(nchu > 0)
    def _():
        pltpu.async_copy(Xin.at[jlist.at[pl.ds(0, G)]], ubuf0, gsem0)

    @pl.when(nchu > 1)
    def _():
        pltpu.async_copy(Xin.at[jlist.at[pl.ds(G, G)]], ubuf1, gsem1)

    def update_chunk(i, _):
        def turn(buf, gsem, ssem):
            pltpu.make_async_copy(Xin.at[jlist.at[pl.ds(i * G, G)]],
                                  buf, gsem).wait()
            pltpu.async_copy(buf, outX.at[dlist2.at[i]], ssem)
            pltpu.make_async_copy(buf, outX.at[dlist2.at[i]], ssem).wait()

            @pl.when(i + 2 < nchu)
            def _():
                pltpu.async_copy(Xin.at[jlist.at[pl.ds((i + 2) * G, G)]],
                                 buf, gsem)

        @pl.when(i % 2 == 0)
        def _():
            turn(ubuf0, gsem0, ssem0)

        @pl.when(i % 2 == 1)
        def _():
            turn(ubuf1, gsem1, ssem1)

        return 0
    lax.fori_loop(0, nchu, update_chunk, 0)

    nch16 = (cnt + L - 1) // L

    def yt_chunk(i, _):
        jv = jlist[pl.ds(i * L, L)]
        dv = dlist[pl.ds(i * L, L)]
        ldv = dv - lo
        plsc.store_scatter(yrange, [ldv], plsc.load_gather(yall, [jv]))
        plsc.store_scatter(trange, [ldv], plsc.load_gather(tall, [jv]))
        return 0
    lax.fori_loop(0, nch16, yt_chunk, 0)

    @pl.when(jnp.logical_not(is_last))
    def _():
        pltpu.sync_copy(yrange.at[pl.ds(0, R)], outy.at[pl.ds(lo, R)])
        pltpu.sync_copy(trange.at[pl.ds(0, R)], outt.at[pl.ds(lo, R)])

    @pl.when(is_last)
    def _():
        pltpu.sync_copy(yrange.at[pl.ds(0, LAST_R)], outy.at[pl.ds(lo, LAST_R)])
        pltpu.sync_copy(trange.at[pl.ds(0, LAST_R)], outt.at[pl.ds(lo, LAST_R)])


_sc_call = functools.partial(
    pl.kernel,
    out_type=(
        jax.ShapeDtypeStruct((B, D), jnp.float32),
        jax.ShapeDtypeStruct((B,), jnp.float32),
        jax.ShapeDtypeStruct((B,), jnp.int32),
    ),
    mesh=plsc.VectorSubcoreMesh(core_axis_name="c", subcore_axis_name="s"),
    compiler_params=pltpu.CompilerParams(needs_layout_passes=False),
    scratch_types=[
        pltpu.VMEM((N,), jnp.int32),
        pltpu.VMEM((R,), jnp.int32),
        pltpu.VMEM((L,), jnp.int32),
        pltpu.VMEM((CAP,), jnp.int32),
        pltpu.VMEM((CAP,), jnp.int32),
        pltpu.VMEM((CAP // G, G), jnp.int32),
        pltpu.VMEM_SHARED((NS, NBUF, G, D), jnp.float32),
        pltpu.VMEM((G, D), jnp.float32),
        pltpu.VMEM((G, D), jnp.float32),
        pltpu.VMEM((N,), jnp.float32),
        pltpu.VMEM((N,), jnp.int32),
        pltpu.VMEM((R,), jnp.float32),
        pltpu.VMEM((R,), jnp.int32),
        pltpu.SemaphoreType.DMA,
        pltpu.SemaphoreType.DMA,
        pltpu.SemaphoreType.DMA,
        pltpu.SemaphoreType.DMA,
        pltpu.SemaphoreType.DMA,
        pltpu.SemaphoreType.DMA,
        pltpu.SemaphoreType.DMA,
        pltpu.SemaphoreType.DMA,
        pltpu.SemaphoreType.DMA,
    ],
)(_body)


def kernel(mem_X, mem_y, mem_task_ids, X, y, task_ids, inds):
    out_X, out_y, out_t = _sc_call(
        mem_X.reshape(B, D), mem_y, mem_task_ids,
        X.reshape(N, D), y, task_ids.astype(jnp.int32),
        inds.astype(jnp.int32))
    return (out_X.reshape(mem_X.shape), out_y, out_t)


# speedup vs baseline: 17.7780x; 1.3558x over previous
import functools

import jax
import jax.numpy as jnp
from jax import lax
from jax.experimental import pallas as pl
from jax.experimental.pallas import tpu as pltpu
from jax.experimental.pallas import tpu_sc as plsc

B = 50000
D = 3072
N = 1024
NC, NS, L = 2, 16, 16
NW = NC * NS
R = 1568
LAST_R = B - (NW - 1) * R
NCHUNK = N // L
CAP = N + L
G = 8


def _body(memy, memt, Xin, yin, tin, inds, outX, outy, outt,
          inds_v, table, s16, jlist, dlist, dlist2, ubuf0, ubuf1,
          yall, tall, yrange, trange,
          gsem0, gsem1, ssem0, ssem1):
    sid = lax.axis_index("s")
    wid = sid * NC + lax.axis_index("c")
    lo = wid * R
    is_last = wid == NW - 1
    iota = lax.iota(jnp.int32, L)

    pltpu.sync_copy(inds, inds_v)
    pltpu.sync_copy(yin, yall)
    pltpu.sync_copy(tin, tall)

    @pl.when(jnp.logical_not(is_last))
    def _():
        pltpu.sync_copy(memy.at[pl.ds(lo, R)], yrange.at[pl.ds(0, R)])
        pltpu.sync_copy(memt.at[pl.ds(lo, R)], trange.at[pl.ds(0, R)])

    @pl.when(is_last)
    def _():
        pltpu.sync_copy(memy.at[pl.ds(lo, LAST_R)], yrange.at[pl.ds(0, LAST_R)])
        pltpu.sync_copy(memt.at[pl.ds(lo, LAST_R)], trange.at[pl.ds(0, LAST_R)])

    def init_tab(i, _):
        table[pl.ds(i * L, L)] = jnp.full((L,), -1, jnp.int32)
        return 0
    lax.fori_loop(0, R // L, init_tab, 0)

    def pass_a(c, _):
        iv = inds_v[pl.ds(c * L, L)]
        jv = iota + c * L
        s16[...] = iv
        dup = jnp.zeros((L,), jnp.int32)
        for s in range(1, L):
            nxt = plsc.load_gather(s16, [jnp.minimum(iota + s, L - 1)])
            valid = (iota + s) <= (L - 1)
            dup = jnp.where(jnp.logical_and(valid, nxt == iv), 1, dup)
        winlane = dup == 0
        local = iv - lo
        inr = jnp.logical_and(local >= 0, local < R)
        localc = jnp.clip(local, 0, R - 1)
        plsc.store_scatter(table, [localc], jv,
                           mask=jnp.logical_and(winlane, inr))
        return 0
    lax.fori_loop(0, NCHUNK, pass_a, 0)

    def pass_b(c, cnt):
        iv = inds_v[pl.ds(c * L, L)]
        jv = iota + c * L
        local = iv - lo
        inr = jnp.logical_and(local >= 0, local < R)
        localc = jnp.clip(local, 0, R - 1)
        tv = plsc.load_gather(table, [localc])
        win = jnp.logical_and(inr, tv == jv)
        wc = plsc.cumsum(win.astype(jnp.int32))
        pos = jnp.clip(cnt + wc - 1, 0, CAP - 1)
        plsc.store_scatter(jlist, [pos], jv, mask=win)
        plsc.store_scatter(dlist, [pos], iv, mask=win)
        plsc.store_scatter(dlist2, [pos // G, pos - (pos // G) * G], iv,
                           mask=win)
        return cnt + jnp.max(wc)
    cnt = lax.fori_loop(0, NCHUNK, pass_b, jnp.int32(0))

    lastp = jnp.full((L,), jnp.clip(cnt - 1, 0, CAP - 1), jnp.int32)
    jlast = plsc.load_gather(jlist, [lastp])
    dlast = plsc.load_gather(dlist, [lastp])
    padp = jnp.clip(cnt + iota, 0, CAP - 1)
    plsc.store_scatter(jlist, [padp], jlast)
    plsc.store_scatter(dlist, [padp], dlast)
    plsc.store_scatter(dlist2, [padp // G, padp - (padp // G) * G], dlast)

    nchu = (cnt + G - 1) // G

    @pl.when(nchu > 0)
    def _():
        pltpu.async_copy(Xin.at[jlist.at[pl.ds(0, G)]], ubuf0, gsem0)

    @pl.when(nchu > 1)
    def _():
        pltpu.async_copy(Xin.at[jlist.at[pl.ds(G, G)]], ubuf1, gsem1)

    def update_chunk(i, _):
        def turn(buf, gsem, ssem):
            pltpu.make_async_copy(Xin.at[jlist.at[pl.ds(i * G, G)]],
                                  buf, gsem).wait()
            pltpu.async_copy(buf, outX.at[dlist2.at[i]], ssem)
            pltpu.make_async_copy(buf, outX.at[dlist2.at[i]], ssem).wait()

            @pl.when(i + 2 < nchu)
            def _():
                pltpu.async_copy(Xin.at[jlist.at[pl.ds((i + 2) * G, G)]],
                                 buf, gsem)

        @pl.when(i % 2 == 0)
        def _():
            turn(ubuf0, gsem0, ssem0)

        @pl.when(i % 2 == 1)
        def _():
            turn(ubuf1, gsem1, ssem1)

        return 0
    lax.fori_loop(0, nchu, update_chunk, 0)

    nch16 = (cnt + L - 1) // L

    def yt_chunk(i, _):
        jv = jlist[pl.ds(i * L, L)]
        dv = dlist[pl.ds(i * L, L)]
        ldv = dv - lo
        plsc.store_scatter(yrange, [ldv], plsc.load_gather(yall, [jv]))
        plsc.store_scatter(trange, [ldv], plsc.load_gather(tall, [jv]))
        return 0
    lax.fori_loop(0, nch16, yt_chunk, 0)

    @pl.when(jnp.logical_not(is_last))
    def _():
        pltpu.sync_copy(yrange.at[pl.ds(0, R)], outy.at[pl.ds(lo, R)])
        pltpu.sync_copy(trange.at[pl.ds(0, R)], outt.at[pl.ds(lo, R)])

    @pl.when(is_last)
    def _():
        pltpu.sync_copy(yrange.at[pl.ds(0, LAST_R)], outy.at[pl.ds(lo, LAST_R)])
        pltpu.sync_copy(trange.at[pl.ds(0, LAST_R)], outt.at[pl.ds(lo, LAST_R)])


_sc_call = functools.partial(
    pl.kernel,
    out_type=(
        jax.ShapeDtypeStruct((B,), jnp.float32),
        jax.ShapeDtypeStruct((B,), jnp.int32),
    ),
    mesh=plsc.VectorSubcoreMesh(core_axis_name="c", subcore_axis_name="s"),
    compiler_params=pltpu.CompilerParams(needs_layout_passes=False),
    scratch_types=[
        pltpu.VMEM((N,), jnp.int32),
        pltpu.VMEM((R,), jnp.int32),
        pltpu.VMEM((L,), jnp.int32),
        pltpu.VMEM((CAP,), jnp.int32),
        pltpu.VMEM((CAP,), jnp.int32),
        pltpu.VMEM((CAP // G, G), jnp.int32),
        pltpu.VMEM((G, D), jnp.float32),
        pltpu.VMEM((G, D), jnp.float32),
        pltpu.VMEM((N,), jnp.float32),
        pltpu.VMEM((N,), jnp.int32),
        pltpu.VMEM((R,), jnp.float32),
        pltpu.VMEM((R,), jnp.int32),
        pltpu.SemaphoreType.DMA,
        pltpu.SemaphoreType.DMA,
        pltpu.SemaphoreType.DMA,
        pltpu.SemaphoreType.DMA,
    ],
)(_body)


def kernel(mem_X, mem_y, mem_task_ids, X, y, task_ids, inds):
    xref = jax.new_ref(mem_X.reshape(B, D))
    out_y, out_t = _sc_call(
        mem_y, mem_task_ids, X.reshape(N, D), y,
        task_ids.astype(jnp.int32), inds.astype(jnp.int32), xref)
    out_X = jax.freeze(xref)
    return (out_X.reshape(mem_X.shape), out_y, out_t)


# speedup vs baseline: 17.8616x; 1.0047x over previous
import functools

import jax
import jax.numpy as jnp
from jax import lax
from jax.experimental import pallas as pl
from jax.experimental.pallas import tpu as pltpu
from jax.experimental.pallas import tpu_sc as plsc

B = 50000
D = 3072
N = 1024
NC, NS, L = 2, 16, 16
NW = NC * NS
R = 1568
LAST_R = B - (NW - 1) * R
NCHUNK = N // L
CAP = N + L
G = 8
_MESH = plsc.VectorSubcoreMesh(core_axis_name="c", subcore_axis_name="s")
_PARAMS = pltpu.CompilerParams(needs_layout_passes=False)


def _route_body(memy, memt, yin, tin, inds, outy, outt, jl_out, dl2_out,
                cnt_out, inds_v, table, s16, jlist, dlist, dlist2, cntbuf,
                yall, tall, yrange, trange):
    sid = lax.axis_index("s")
    wid = sid * NC + lax.axis_index("c")
    lo = wid * R
    is_last = wid == NW - 1
    iota = lax.iota(jnp.int32, L)

    pltpu.sync_copy(inds, inds_v)
    pltpu.sync_copy(yin, yall)
    pltpu.sync_copy(tin, tall)

    @pl.when(jnp.logical_not(is_last))
    def _():
        pltpu.sync_copy(memy.at[pl.ds(lo, R)], yrange.at[pl.ds(0, R)])
        pltpu.sync_copy(memt.at[pl.ds(lo, R)], trange.at[pl.ds(0, R)])

    @pl.when(is_last)
    def _():
        pltpu.sync_copy(memy.at[pl.ds(lo, LAST_R)], yrange.at[pl.ds(0, LAST_R)])
        pltpu.sync_copy(memt.at[pl.ds(lo, LAST_R)], trange.at[pl.ds(0, LAST_R)])

    def init_tab(i, _):
        table[pl.ds(i * L, L)] = jnp.full((L,), -1, jnp.int32)
        return 0
    lax.fori_loop(0, R // L, init_tab, 0)

    def pass_a(c, _):
        iv = inds_v[pl.ds(c * L, L)]
        jv = iota + c * L
        s16[...] = iv
        dup = jnp.zeros((L,), jnp.int32)
        for s in range(1, L):
            nxt = plsc.load_gather(s16, [jnp.minimum(iota + s, L - 1)])
            valid = (iota + s) <= (L - 1)
            dup = jnp.where(jnp.logical_and(valid, nxt == iv), 1, dup)
        winlane = dup == 0
        local = iv - lo
        inr = jnp.logical_and(local >= 0, local < R)
        localc = jnp.clip(local, 0, R - 1)
        plsc.store_scatter(table, [localc], jv,
                           mask=jnp.logical_and(winlane, inr))
        return 0
    lax.fori_loop(0, NCHUNK, pass_a, 0)

    def pass_b(c, cnt):
        iv = inds_v[pl.ds(c * L, L)]
        jv = iota + c * L
        local = iv - lo
        inr = jnp.logical_and(local >= 0, local < R)
        localc = jnp.clip(local, 0, R - 1)
        tv = plsc.load_gather(table, [localc])
        win = jnp.logical_and(inr, tv == jv)
        wc = plsc.cumsum(win.astype(jnp.int32))
        pos = jnp.clip(cnt + wc - 1, 0, CAP - 1)
        plsc.store_scatter(jlist, [pos], jv, mask=win)
        plsc.store_scatter(dlist, [pos], iv, mask=win)
        plsc.store_scatter(dlist2, [pos // G, pos - (pos // G) * G], iv,
                           mask=win)
        return cnt + jnp.max(wc)
    cnt = lax.fori_loop(0, NCHUNK, pass_b, jnp.int32(0))

    lastp = jnp.full((L,), jnp.clip(cnt - 1, 0, CAP - 1), jnp.int32)
    jlast = plsc.load_gather(jlist, [lastp])
    dlast = plsc.load_gather(dlist, [lastp])
    padp = jnp.clip(cnt + iota, 0, CAP - 1)
    plsc.store_scatter(jlist, [padp], jlast)
    plsc.store_scatter(dlist, [padp], dlast)
    plsc.store_scatter(dlist2, [padp // G, padp - (padp // G) * G], dlast)

    cntbuf[...] = jnp.full((L,), cnt, jnp.int32)
    pltpu.sync_copy(jlist, jl_out.at[wid])
    pltpu.sync_copy(dlist2, dl2_out.at[wid])
    pltpu.sync_copy(cntbuf, cnt_out.at[wid])

    nch16 = (cnt + L - 1) // L

    def yt_chunk(i, _):
        jv = jlist[pl.ds(i * L, L)]
        dv = dlist[pl.ds(i * L, L)]
        ldv = dv - lo
        plsc.store_scatter(yrange, [ldv], plsc.load_gather(yall, [jv]))
        plsc.store_scatter(trange, [ldv], plsc.load_gather(tall, [jv]))
        return 0
    lax.fori_loop(0, nch16, yt_chunk, 0)

    @pl.when(jnp.logical_not(is_last))
    def _():
        pltpu.sync_copy(yrange.at[pl.ds(0, R)], outy.at[pl.ds(lo, R)])
        pltpu.sync_copy(trange.at[pl.ds(0, R)], outt.at[pl.ds(lo, R)])

    @pl.when(is_last)
    def _():
        pltpu.sync_copy(yrange.at[pl.ds(0, LAST_R)], outy.at[pl.ds(lo, LAST_R)])
        pltpu.sync_copy(trange.at[pl.ds(0, LAST_R)], outt.at[pl.ds(lo, LAST_R)])


_route_call = functools.partial(
    pl.kernel,
    out_type=(
        jax.ShapeDtypeStruct((B,), jnp.float32),
        jax.ShapeDtypeStruct((B,), jnp.int32),
        jax.ShapeDtypeStruct((NW, CAP), jnp.int32),
        jax.ShapeDtypeStruct((NW, CAP // G, G), jnp.int32),
        jax.ShapeDtypeStruct((NW, L), jnp.int32),
    ),
    mesh=_MESH,
    compiler_params=_PARAMS,
    scratch_types=[
        pltpu.VMEM((N,), jnp.int32),
        pltpu.VMEM((R,), jnp.int32),
        pltpu.VMEM((L,), jnp.int32),
        pltpu.VMEM((CAP,), jnp.int32),
        pltpu.VMEM((CAP,), jnp.int32),
        pltpu.VMEM((CAP // G, G), jnp.int32),
        pltpu.VMEM((L,), jnp.int32),
        pltpu.VMEM((N,), jnp.float32),
        pltpu.VMEM((N,), jnp.int32),
        pltpu.VMEM((R,), jnp.float32),
        pltpu.VMEM((R,), jnp.int32),
    ],
)(_route_body)


def _scatter_body(Xin, jl_in, dl2_in, cnt_in, outX,
                  jlist, dlist2, cntbuf, ubuf0, ubuf1,
                  gsem0, gsem1, ssem0, ssem1):
    sid = lax.axis_index("s")
    wid = sid * NC + lax.axis_index("c")

    pltpu.sync_copy(jl_in.at[wid], jlist)
    pltpu.sync_copy(dl2_in.at[wid], dlist2)
    pltpu.sync_copy(cnt_in.at[wid], cntbuf)
    cnt = jnp.max(cntbuf[...])

    nchu = (cnt + G - 1) // G

    @pl.when(nchu > 0)
    def _():
        pltpu.async_copy(Xin.at[jlist.at[pl.ds(0, G)]], ubuf0, gsem0)

    @pl.when(nchu > 1)
    def _():
        pltpu.async_copy(Xin.at[jlist.at[pl.ds(G, G)]], ubuf1, gsem1)

    def update_chunk(i, _):
        def turn(buf, gsem, ssem):
            pltpu.make_async_copy(Xin.at[jlist.at[pl.ds(i * G, G)]],
                                  buf, gsem).wait()
            pltpu.async_copy(buf, outX.at[dlist2.at[i]], ssem)
            pltpu.make_async_copy(buf, outX.at[dlist2.at[i]], ssem).wait()

            @pl.when(i + 2 < nchu)
            def _():
                pltpu.async_copy(Xin.at[jlist.at[pl.ds((i + 2) * G, G)]],
                                 buf, gsem)

        @pl.when(i % 2 == 0)
        def _():
            turn(ubuf0, gsem0, ssem0)

        @pl.when(i % 2 == 1)
        def _():
            turn(ubuf1, gsem1, ssem1)

        return 0
    lax.fori_loop(0, nchu, update_chunk, 0)


_scatter_call = functools.partial(
    pl.kernel,
    out_type=(),
    mesh=_MESH,
    compiler_params=_PARAMS,
    scratch_types=[
        pltpu.VMEM((CAP,), jnp.int32),
        pltpu.VMEM((CAP // G, G), jnp.int32),
        pltpu.VMEM((L,), jnp.int32),
        pltpu.VMEM((G, D), jnp.float32),
        pltpu.VMEM((G, D), jnp.float32),
        pltpu.SemaphoreType.DMA,
        pltpu.SemaphoreType.DMA,
        pltpu.SemaphoreType.DMA,
        pltpu.SemaphoreType.DMA,
    ],
)(_scatter_body)


def kernel(mem_X, mem_y, mem_task_ids, X, y, task_ids, inds):
    xref = jax.new_ref(mem_X.reshape(B, D))
    out_y, out_t, jl, dl2, cnts = _route_call(
        mem_y, mem_task_ids, y, task_ids.astype(jnp.int32),
        inds.astype(jnp.int32))
    _scatter_call(X.reshape(N, D), jl, dl2, cnts, xref)
    out_X = jax.freeze(xref)
    return (out_X.reshape(mem_X.shape), out_y, out_t)
